# baseline (speedup 1.0000x reference)
import functools

import jax
import jax.numpy as jnp
from jax import lax
from jax.experimental import pallas as pl
from jax.experimental.pallas import tpu as pltpu
from jax.experimental.pallas import tpu_sc as plsc

NUM_FIELDS = 26
VOCAB = 100000
EMBED_DIM = 32
BATCH = 4096

NC, NS, L = 2, 16, 16
NW = NC * NS
TOTAL = BATCH * NUM_FIELDS
PER_W = TOTAL // NW


def _multi_embed(x_flat, table_flat):
    mesh = plsc.VectorSubcoreMesh(core_axis_name="c", subcore_axis_name="s")

    @functools.partial(
        pl.kernel,
        mesh=mesh,
        out_type=jax.ShapeDtypeStruct((TOTAL, EMBED_DIM), jnp.float32),
        scratch_types=[
            pltpu.VMEM((PER_W,), jnp.int32),
            pltpu.VMEM((PER_W, EMBED_DIM), jnp.float32),
            pltpu.SemaphoreType.DMA,
        ],
        compiler_params=pltpu.CompilerParams(use_tc_tiling_on_sc=False),
    )
    def k(idx_hbm, table_hbm, out_hbm, idx_v, rows_v, sem):
        wid = lax.axis_index("s") * NC + lax.axis_index("c")
        base = wid * PER_W
        pltpu.sync_copy(idx_hbm.at[pl.ds(base, PER_W)], idx_v)

        lane = lax.iota(jnp.int32, L)

        def add_offsets(i, carry):
            sl = pl.ds(i * L, L)
            pos = lane + i * L
            field = lax.rem(pos, jnp.int32(NUM_FIELDS))
            idx_v[sl] = idx_v[sl] + field * VOCAB
            return carry

        lax.fori_loop(0, PER_W // L, add_offsets, 0)

        pltpu.async_copy(table_hbm.at[idx_v], rows_v, sem).wait()
        pltpu.sync_copy(rows_v, out_hbm.at[pl.ds(base, PER_W)])

    return k(x_flat, table_flat)


def kernel(x, tables):
    out = _multi_embed(
        x.reshape(TOTAL),
        tables.reshape(NUM_FIELDS * VOCAB, EMBED_DIM),
    )
    return out.reshape(BATCH, NUM_FIELDS * EMBED_DIM)


# speedup vs baseline: 6.2125x; 6.2125x over previous
import functools

import jax
import jax.numpy as jnp
from jax import lax
from jax.experimental import pallas as pl
from jax.experimental.pallas import tpu as pltpu
from jax.experimental.pallas import tpu_sc as plsc

NUM_FIELDS = 26
VOCAB = 100000
EMBED_DIM = 32
BATCH = 4096

NC, NS, L = 2, 16, 16
NW = NC * NS
NQ = NUM_FIELDS * EMBED_DIM
PER_W = NQ // NW


def _multi_embed(x_t, tbl2):
    mesh = plsc.VectorSubcoreMesh(core_axis_name="c", subcore_axis_name="s")

    @functools.partial(
        pl.kernel,
        mesh=mesh,
        out_type=jax.ShapeDtypeStruct((NQ, BATCH), jnp.float32),
        scratch_types=[
            pltpu.VMEM((VOCAB,), jnp.float32),
            pltpu.VMEM((BATCH,), jnp.int32),
            pltpu.VMEM((BATCH,), jnp.float32),
            pltpu.SemaphoreType.DMA,
        ],
        compiler_params=pltpu.CompilerParams(
            use_tc_tiling_on_sc=True, needs_layout_passes=False
        ),
    )
    def k(xt_hbm, tbl_hbm, out_hbm, tblrow_v, idx_v, row_v, sem):
        wid = lax.axis_index("s") * NC + lax.axis_index("c")
        q0 = wid * PER_W

        def task(i, carry):
            q = q0 + i
            f = lax.div(q, jnp.int32(EMBED_DIM))
            cp_idx = pltpu.async_copy(xt_hbm.at[f], idx_v, sem)
            cp_row = pltpu.async_copy(tbl_hbm.at[q], tblrow_v, sem)
            cp_idx.wait()
            cp_row.wait()

            def gath(j, c2):
                sl = pl.ds(j * L, L)
                row_v[sl] = plsc.load_gather(tblrow_v, [idx_v[sl]])
                return c2

            lax.fori_loop(0, BATCH // L, gath, 0)
            pltpu.sync_copy(row_v, out_hbm.at[q])
            return carry

        lax.fori_loop(0, PER_W, task, 0)

    return k(x_t, tbl2)


def kernel(x, tables):
    tbl2 = tables.transpose(0, 2, 1).reshape(NQ, VOCAB)
    out_t = _multi_embed(x.T, tbl2)
    return out_t.T


# speedup vs baseline: 7.6618x; 1.2333x over previous
import functools

import jax
import jax.numpy as jnp
from jax import lax
from jax.experimental import pallas as pl
from jax.experimental.pallas import tpu as pltpu
from jax.experimental.pallas import tpu_sc as plsc

NUM_FIELDS = 26
VOCAB = 100000
EMBED_DIM = 32
BATCH = 4096

NC, NS, L = 2, 16, 16
NW = NC * NS
NQ = NUM_FIELDS * EMBED_DIM
PER_W = NQ // NW


def _multi_embed(x_t, tbl2):
    mesh = plsc.VectorSubcoreMesh(core_axis_name="c", subcore_axis_name="s")

    @functools.partial(
        pl.kernel,
        mesh=mesh,
        out_type=jax.ShapeDtypeStruct((NQ, BATCH), jnp.float32),
        scratch_types=[
            pltpu.VMEM((VOCAB,), jnp.float32),
            pltpu.VMEM((BATCH,), jnp.int32),
            pltpu.VMEM((BATCH,), jnp.float32),
            pltpu.SemaphoreType.DMA,
        ],
        compiler_params=pltpu.CompilerParams(
            use_tc_tiling_on_sc=True, needs_layout_passes=False
        ),
    )
    def k(xt_hbm, tbl_hbm, out_hbm, tblrow_v, idx_v, row_v, sem):
        wid = lax.axis_index("s") * NC + lax.axis_index("c")
        q0 = wid * PER_W

        def task(i, f_prev):
            q = q0 + i
            f = lax.div(q, jnp.int32(EMBED_DIM))
            cp_row = pltpu.async_copy(tbl_hbm.at[q], tblrow_v, sem)

            @pl.when(f != f_prev)
            def _():
                pltpu.sync_copy(xt_hbm.at[f], idx_v)

            cp_row.wait()

            @plsc.parallel_loop(0, BATCH // L, unroll=8)
            def gath(j):
                sl = pl.ds(j * L, L)
                row_v[sl] = plsc.load_gather(tblrow_v, [idx_v[sl]])

            pltpu.sync_copy(row_v, out_hbm.at[q])
            return f

        lax.fori_loop(0, PER_W, task, jnp.int32(-1))

    return k(x_t, tbl2)


def kernel(x, tables):
    tbl2 = tables.transpose(0, 2, 1).reshape(NQ, VOCAB)
    out_t = _multi_embed(x.T, tbl2)
    return out_t.T
